# Initial kernel scaffold; baseline (speedup 1.0000x reference)
import jax
import jax.numpy as jnp
from jax.experimental import pallas as pl


def kernel(pre_spikes, post_spikes, weights, pre_trace, post_trace, theta, x_meta, current_time):
    raise NotImplementedError("write your pallas kernel here")


# speedup vs baseline: 6.0401x; 6.0401x over previous
import functools

import jax
import jax.numpy as jnp
from jax.experimental import pallas as pl

PRE = 2048
POST = 2048
W_MIN = 0.0
W_MAX = 1.0
TAU_PLUS = 0.02
TAU_MINUS = 0.02
TAU_THETA = 10.0
TAU_X = 0.1
TARGET_ACTIVITY = 0.1
DT = 0.001

BLK = 256


def _fused_kernel(pre_ref, post_ref, w_ref, pre_tr_in_ref, post_tr_in_ref,
                  theta_ref, x_ref,
                  sc_ref, new_w_ref, pre_tr_ref, post_tr_ref,
                  theta_new_ref, x_new_ref):
    i = pl.program_id(0)

    w = w_ref[...]
    new_w_ref[...] = jnp.clip(w, W_MIN, W_MAX)

    partial = jax.lax.dot_general(
        pre_ref[...], w, (((1,), (0,)), ((), ())),
        preferred_element_type=jnp.float32,
        precision=jax.lax.Precision.HIGHEST)

    @pl.when(i == 0)
    def _init():
        sc_ref[...] = partial
        post = post_ref[...]
        decay_minus = jnp.exp(jnp.float32(-DT / TAU_MINUS))
        post_tr_ref[...] = post_tr_in_ref[...] * decay_minus + post
        decay_x = jnp.exp(jnp.float32(-DT / TAU_X))
        x_new = x_ref[...] * decay_x + jnp.mean(post, axis=0, keepdims=True)
        x_new_ref[...] = x_new
        theta = theta_ref[...]
        theta_new_ref[...] = theta + jnp.float32(DT / TAU_THETA) * (
            x_new * x_new * jnp.float32(1.0 / TARGET_ACTIVITY) - theta)

    @pl.when(i != 0)
    def _acc():
        sc_ref[...] += partial

    decay_plus = jnp.exp(jnp.float32(-DT / TAU_PLUS))
    pre_tr_ref[...] = pre_tr_in_ref[...] * decay_plus + pre_ref[...]


@functools.partial(jax.jit, static_argnames=())
def _run(pre_spikes, post_spikes, weights, pre_trace, post_trace, theta, x_meta):
    b = pre_spikes.shape[0]
    n_blk = PRE // BLK
    grid = (n_blk,)
    out = pl.pallas_call(
        _fused_kernel,
        grid=grid,
        in_specs=[
            pl.BlockSpec((b, BLK), lambda i: (0, i)),
            pl.BlockSpec((b, POST), lambda i: (0, 0)),
            pl.BlockSpec((BLK, POST), lambda i: (i, 0)),
            pl.BlockSpec((1, BLK), lambda i: (0, i)),
            pl.BlockSpec((1, POST), lambda i: (0, 0)),
            pl.BlockSpec((1, POST), lambda i: (0, 0)),
            pl.BlockSpec((1, POST), lambda i: (0, 0)),
        ],
        out_specs=[
            pl.BlockSpec((b, POST), lambda i: (0, 0)),
            pl.BlockSpec((BLK, POST), lambda i: (i, 0)),
            pl.BlockSpec((b, BLK), lambda i: (0, i)),
            pl.BlockSpec((b, POST), lambda i: (0, 0)),
            pl.BlockSpec((1, POST), lambda i: (0, 0)),
            pl.BlockSpec((1, POST), lambda i: (0, 0)),
        ],
        out_shape=[
            jax.ShapeDtypeStruct((b, POST), jnp.float32),
            jax.ShapeDtypeStruct((PRE, POST), jnp.float32),
            jax.ShapeDtypeStruct((b, PRE), jnp.float32),
            jax.ShapeDtypeStruct((b, POST), jnp.float32),
            jax.ShapeDtypeStruct((1, POST), jnp.float32),
            jax.ShapeDtypeStruct((1, POST), jnp.float32),
        ],
    )(pre_spikes, post_spikes, weights,
      pre_trace.reshape(1, PRE), post_trace.reshape(1, POST),
      theta.reshape(1, POST), x_meta.reshape(1, POST))
    sc, new_w, pre_tr, post_tr, theta_new, x_new = out
    return (sc, new_w, pre_tr, post_tr,
            theta_new.reshape(POST), x_new.reshape(POST))


def kernel(pre_spikes, post_spikes, weights, pre_trace, post_trace, theta,
           x_meta, current_time):
    return _run(pre_spikes, post_spikes, weights, pre_trace, post_trace,
                theta, x_meta)


# speedup vs baseline: 7.1952x; 1.1912x over previous
import functools

import jax
import jax.numpy as jnp
from jax.experimental import pallas as pl

PRE = 2048
POST = 2048
W_MIN = 0.0
W_MAX = 1.0
TAU_PLUS = 0.02
TAU_MINUS = 0.02
TAU_THETA = 10.0
TAU_X = 0.1
TARGET_ACTIVITY = 0.1
DT = 0.001

BLK = 256


def _fused_kernel(pre_ref, post_ref, w_ref, pre_tr_in_ref, post_tr_in_ref,
                  theta_ref, x_ref,
                  sc_ref, new_w_ref, pre_tr_ref, post_tr_ref,
                  theta_new_ref, x_new_ref):
    i = pl.program_id(0)

    w = w_ref[...]
    new_w_ref[...] = jnp.clip(w, W_MIN, W_MAX)

    partial = jax.lax.dot_general(
        pre_ref[...], w, (((1,), (0,)), ((), ())),
        preferred_element_type=jnp.float32,
        precision=jax.lax.Precision.DEFAULT)

    @pl.when(i == 0)
    def _init():
        sc_ref[...] = partial
        post = post_ref[...]
        decay_minus = jnp.exp(jnp.float32(-DT / TAU_MINUS))
        post_tr_ref[...] = post_tr_in_ref[...] * decay_minus + post
        decay_x = jnp.exp(jnp.float32(-DT / TAU_X))
        x_new = x_ref[...] * decay_x + jnp.mean(post, axis=0, keepdims=True)
        x_new_ref[...] = x_new
        theta = theta_ref[...]
        theta_new_ref[...] = theta + jnp.float32(DT / TAU_THETA) * (
            x_new * x_new * jnp.float32(1.0 / TARGET_ACTIVITY) - theta)

    @pl.when(i != 0)
    def _acc():
        sc_ref[...] += partial

    decay_plus = jnp.exp(jnp.float32(-DT / TAU_PLUS))
    pre_tr_ref[...] = pre_tr_in_ref[...] * decay_plus + pre_ref[...]


@functools.partial(jax.jit, static_argnames=())
def _run(pre_spikes, post_spikes, weights, pre_trace, post_trace, theta, x_meta):
    b = pre_spikes.shape[0]
    n_blk = PRE // BLK
    grid = (n_blk,)
    out = pl.pallas_call(
        _fused_kernel,
        grid=grid,
        in_specs=[
            pl.BlockSpec((b, BLK), lambda i: (0, i)),
            pl.BlockSpec((b, POST), lambda i: (0, 0)),
            pl.BlockSpec((BLK, POST), lambda i: (i, 0)),
            pl.BlockSpec((1, BLK), lambda i: (0, i)),
            pl.BlockSpec((1, POST), lambda i: (0, 0)),
            pl.BlockSpec((1, POST), lambda i: (0, 0)),
            pl.BlockSpec((1, POST), lambda i: (0, 0)),
        ],
        out_specs=[
            pl.BlockSpec((b, POST), lambda i: (0, 0)),
            pl.BlockSpec((BLK, POST), lambda i: (i, 0)),
            pl.BlockSpec((b, BLK), lambda i: (0, i)),
            pl.BlockSpec((b, POST), lambda i: (0, 0)),
            pl.BlockSpec((1, POST), lambda i: (0, 0)),
            pl.BlockSpec((1, POST), lambda i: (0, 0)),
        ],
        out_shape=[
            jax.ShapeDtypeStruct((b, POST), jnp.float32),
            jax.ShapeDtypeStruct((PRE, POST), jnp.float32),
            jax.ShapeDtypeStruct((b, PRE), jnp.float32),
            jax.ShapeDtypeStruct((b, POST), jnp.float32),
            jax.ShapeDtypeStruct((1, POST), jnp.float32),
            jax.ShapeDtypeStruct((1, POST), jnp.float32),
        ],
    )(pre_spikes, post_spikes, weights,
      pre_trace.reshape(1, PRE), post_trace.reshape(1, POST),
      theta.reshape(1, POST), x_meta.reshape(1, POST))
    sc, new_w, pre_tr, post_tr, theta_new, x_new = out
    return (sc, new_w, pre_tr, post_tr,
            theta_new.reshape(POST), x_new.reshape(POST))


def kernel(pre_spikes, post_spikes, weights, pre_trace, post_trace, theta,
           x_meta, current_time):
    return _run(pre_spikes, post_spikes, weights, pre_trace, post_trace,
                theta, x_meta)


# speedup vs baseline: 7.4436x; 1.0345x over previous
import functools

import jax
import jax.numpy as jnp
from jax.experimental import pallas as pl

PRE = 2048
POST = 2048
W_MIN = 0.0
W_MAX = 1.0
TAU_PLUS = 0.02
TAU_MINUS = 0.02
TAU_THETA = 10.0
TAU_X = 0.1
TARGET_ACTIVITY = 0.1
DT = 0.001

BLK = 512


def _fused_kernel(pre_ref, post_ref, w_ref, pre_tr_in_ref, post_tr_in_ref,
                  theta_ref, x_ref,
                  sc_ref, new_w_ref, pre_tr_ref, post_tr_ref,
                  theta_new_ref, x_new_ref):
    i = pl.program_id(0)

    w = w_ref[...]
    new_w_ref[...] = jnp.clip(w, W_MIN, W_MAX)

    partial = jax.lax.dot_general(
        pre_ref[...], w, (((1,), (0,)), ((), ())),
        preferred_element_type=jnp.float32,
        precision=jax.lax.Precision.DEFAULT)

    @pl.when(i == 0)
    def _init():
        sc_ref[...] = partial
        post = post_ref[...]
        decay_minus = jnp.exp(jnp.float32(-DT / TAU_MINUS))
        post_tr_ref[...] = post_tr_in_ref[...] * decay_minus + post
        decay_x = jnp.exp(jnp.float32(-DT / TAU_X))
        x_new = x_ref[...] * decay_x + jnp.mean(post, axis=0, keepdims=True)
        x_new_ref[...] = x_new
        theta = theta_ref[...]
        theta_new_ref[...] = theta + jnp.float32(DT / TAU_THETA) * (
            x_new * x_new * jnp.float32(1.0 / TARGET_ACTIVITY) - theta)

    @pl.when(i != 0)
    def _acc():
        sc_ref[...] += partial

    decay_plus = jnp.exp(jnp.float32(-DT / TAU_PLUS))
    pre_tr_ref[...] = pre_tr_in_ref[...] * decay_plus + pre_ref[...]


@functools.partial(jax.jit, static_argnames=())
def _run(pre_spikes, post_spikes, weights, pre_trace, post_trace, theta, x_meta):
    b = pre_spikes.shape[0]
    n_blk = PRE // BLK
    grid = (n_blk,)
    out = pl.pallas_call(
        _fused_kernel,
        grid=grid,
        in_specs=[
            pl.BlockSpec((b, BLK), lambda i: (0, i)),
            pl.BlockSpec((b, POST), lambda i: (0, 0)),
            pl.BlockSpec((BLK, POST), lambda i: (i, 0)),
            pl.BlockSpec((1, BLK), lambda i: (0, i)),
            pl.BlockSpec((1, POST), lambda i: (0, 0)),
            pl.BlockSpec((1, POST), lambda i: (0, 0)),
            pl.BlockSpec((1, POST), lambda i: (0, 0)),
        ],
        out_specs=[
            pl.BlockSpec((b, POST), lambda i: (0, 0)),
            pl.BlockSpec((BLK, POST), lambda i: (i, 0)),
            pl.BlockSpec((b, BLK), lambda i: (0, i)),
            pl.BlockSpec((b, POST), lambda i: (0, 0)),
            pl.BlockSpec((1, POST), lambda i: (0, 0)),
            pl.BlockSpec((1, POST), lambda i: (0, 0)),
        ],
        out_shape=[
            jax.ShapeDtypeStruct((b, POST), jnp.float32),
            jax.ShapeDtypeStruct((PRE, POST), jnp.float32),
            jax.ShapeDtypeStruct((b, PRE), jnp.float32),
            jax.ShapeDtypeStruct((b, POST), jnp.float32),
            jax.ShapeDtypeStruct((1, POST), jnp.float32),
            jax.ShapeDtypeStruct((1, POST), jnp.float32),
        ],
    )(pre_spikes, post_spikes, weights,
      pre_trace.reshape(1, PRE), post_trace.reshape(1, POST),
      theta.reshape(1, POST), x_meta.reshape(1, POST))
    sc, new_w, pre_tr, post_tr, theta_new, x_new = out
    return (sc, new_w, pre_tr, post_tr,
            theta_new.reshape(POST), x_new.reshape(POST))


def kernel(pre_spikes, post_spikes, weights, pre_trace, post_trace, theta,
           x_meta, current_time):
    return _run(pre_spikes, post_spikes, weights, pre_trace, post_trace,
                theta, x_meta)
